# baseline (speedup 1.0000x reference)
import functools

import jax
import jax.numpy as jnp
from jax import lax
from jax.experimental import pallas as pl
from jax.experimental.pallas import tpu as pltpu
from jax.experimental.pallas import tpu_sc as plsc

N_NODES = 10000
H = 128
NUM_RELS = 16
E = 320000

NC = 2
NS = 16
NW = NC * NS
CHUNK = 32
K = 16
NG = 20
EPW = NG * K * CHUNK
E_PAD = NW * EPW
NPAD = 10112
RPS = NPAD // NS

NBLK = 400
GRID = N_NODES // NBLK


def _xw_body(coeff_ref, bases_ref, x_ref, xw_ref, wrel):
    @pl.when(pl.program_id(0) == 0)
    def _():
        for r in range(NUM_RELS):
            acc = coeff_ref[r, 0] * bases_ref[0]
            for b in range(1, 4):
                acc = acc + coeff_ref[r, b] * bases_ref[b]
            wrel[r] = acc

    xb = x_ref[...]
    for r in range(NUM_RELS):
        xw_ref[r] = jnp.dot(xb, wrel[r], preferred_element_type=jnp.float32)


_xw_call = pl.pallas_call(
    _xw_body,
    grid=(GRID,),
    in_specs=[
        pl.BlockSpec(memory_space=pltpu.SMEM),
        pl.BlockSpec((4, H, H), lambda i: (0, 0, 0)),
        pl.BlockSpec((NBLK, H), lambda i: (i, 0)),
    ],
    out_specs=pl.BlockSpec((NUM_RELS, NBLK, H), lambda i: (0, i, 0)),
    out_shape=jax.ShapeDtypeStruct((NUM_RELS, N_NODES, H), jnp.float32),
    scratch_shapes=[pltpu.VMEM((NUM_RELS, H, H), jnp.float32)],
)


def _agg_body(xw_hbm, gidx_hbm, didx_hbm, z2_hbm,
              agg_out,
              gidx_v, didx_v, rows0, rows1, rows2, rows3, rows4,
              agg_shared, gsem, ssem):
    c = lax.axis_index("c")
    s = lax.axis_index("s")
    w = c * NS + s
    base = s * RPS
    rows = (rows0, rows1, rows2, rows3, rows4)
    pltpu.sync_copy(z2_hbm.at[pl.ds(base, RPS)], agg_shared.at[pl.ds(base, RPS)])
    plsc.subcore_barrier()

    def group(g, carry):
        pltpu.sync_copy(gidx_hbm.at[w, g], gidx_v)
        pltpu.sync_copy(didx_hbm.at[w, g], didx_v)
        gds = {}
        for p in range(3):
            gds[p] = pltpu.async_copy(xw_hbm.at[gidx_v.at[p]], rows[p], gsem)
        sds = []
        for t in range(K):
            gds[t].wait()
            sds.append(pltpu.async_copy(
                rows[t % 5], agg_shared.at[didx_v.at[t]], ssem, add=True))
            if t >= 2:
                sds[t - 2].wait()
            if t + 3 < K:
                gds[t + 3] = pltpu.async_copy(
                    xw_hbm.at[gidx_v.at[t + 3]], rows[(t + 3) % 5], gsem)
        sds[K - 2].wait()
        sds[K - 1].wait()
        return carry

    lax.fori_loop(0, NG, group, 0)
    plsc.subcore_barrier()
    pltpu.sync_copy(agg_shared.at[pl.ds(base, RPS)],
                    agg_out.at[c, pl.ds(base, RPS)])


@functools.lru_cache(maxsize=1)
def _get_agg_call():
    return functools.partial(
        pl.kernel,
        out_type=jax.ShapeDtypeStruct((NC, NPAD, H), jnp.float32),
        mesh=plsc.VectorSubcoreMesh(core_axis_name="c", subcore_axis_name="s",
                                    num_cores=NC, num_subcores=NS),
        scratch_types=[
            pltpu.VMEM((K, CHUNK), jnp.int32),
            pltpu.VMEM((K, CHUNK), jnp.int32),
            pltpu.VMEM((CHUNK, H), jnp.float32),
            pltpu.VMEM((CHUNK, H), jnp.float32),
            pltpu.VMEM((CHUNK, H), jnp.float32),
            pltpu.VMEM((CHUNK, H), jnp.float32),
            pltpu.VMEM((CHUNK, H), jnp.float32),
            pltpu.VMEM_SHARED((NPAD, H), jnp.float32),
            pltpu.SemaphoreType.DMA,
            pltpu.SemaphoreType.DMA,
        ],
    )(_agg_body)


def _deg_body(didx_hbm, z2_hbm, ones_hbm,
              deg_out,
              didx_v, ones_v, deg_shared, ssem):
    c = lax.axis_index("c")
    s = lax.axis_index("s")
    w = c * NS + s
    base = s * RPS
    pltpu.sync_copy(z2_hbm.at[pl.ds(base, RPS)], deg_shared.at[pl.ds(base, RPS)])
    pltpu.sync_copy(ones_hbm, ones_v)
    plsc.subcore_barrier()

    def group(g, carry):
        pltpu.sync_copy(didx_hbm.at[w, g], didx_v)
        sds = []
        for t in range(K):
            if t >= 8:
                sds[t - 8].wait()
            sds.append(pltpu.async_copy(
                ones_v, deg_shared.at[didx_v.at[t]], ssem, add=True))
        for t in range(max(0, K - 8), K):
            sds[t].wait()
        return carry

    lax.fori_loop(0, NG, group, 0)
    plsc.subcore_barrier()
    pltpu.sync_copy(deg_shared.at[pl.ds(base, RPS)],
                    deg_out.at[c, pl.ds(base, RPS)])


@functools.lru_cache(maxsize=1)
def _get_deg_call():
    return functools.partial(
        pl.kernel,
        out_type=jax.ShapeDtypeStruct((NC, NPAD, H), jnp.float32),
        mesh=plsc.VectorSubcoreMesh(core_axis_name="c", subcore_axis_name="s",
                                    num_cores=NC, num_subcores=NS),
        scratch_types=[
            pltpu.VMEM((K, CHUNK), jnp.int32),
            pltpu.VMEM((CHUNK, H), jnp.float32),
            pltpu.VMEM_SHARED((NPAD, H), jnp.float32),
            pltpu.SemaphoreType.DMA,
        ],
    )(_deg_body)


def _combine_body(agg_ref, deg_ref, x_ref, w_self_ref, out_ref):
    deg = deg_ref[0, :, 0:1] + deg_ref[1, :, 0:1]
    deg = jnp.maximum(deg, 1.0)
    agg = agg_ref[0] + agg_ref[1]
    out_ref[...] = agg / deg + jnp.dot(
        x_ref[...], w_self_ref[...], preferred_element_type=jnp.float32)


_combine_call = pl.pallas_call(
    _combine_body,
    grid=(GRID,),
    in_specs=[
        pl.BlockSpec((NC, NBLK, H), lambda i: (0, i, 0)),
        pl.BlockSpec((NC, NBLK, H), lambda i: (0, i, 0)),
        pl.BlockSpec((NBLK, H), lambda i: (i, 0)),
        pl.BlockSpec((H, H), lambda i: (0, 0)),
    ],
    out_specs=pl.BlockSpec((NBLK, H), lambda i: (i, 0)),
    out_shape=jax.ShapeDtypeStruct((N_NODES, H), jnp.float32),
)


def kernel(h, edge_index, edge_type, embed_table, bases, coeff, w_self):
    x = jnp.take(embed_table, h, axis=0)
    xw = _xw_call(coeff, bases, x)
    xw_flat = xw.reshape(NUM_RELS * N_NODES, H)

    src = edge_index[0]
    dst = edge_index[1]
    flat = edge_type * N_NODES + src
    pad = E_PAD - E
    gidx = jnp.concatenate(
        [flat, jnp.zeros((pad,), jnp.int32)]).reshape(NW, NG, K, CHUNK)
    didx_flat = jnp.concatenate([dst, jnp.full((pad,), N_NODES, jnp.int32)])
    didx = didx_flat.reshape(NW, NG, K, CHUNK)
    z2 = jnp.zeros((NPAD, H), jnp.float32)
    ones_in = jnp.ones((CHUNK, H), jnp.float32)

    agg_part = _get_agg_call()(xw_flat, gidx, didx, z2)
    deg_part = _get_deg_call()(didx, z2, ones_in)
    out = _combine_call(agg_part, deg_part, x, w_self)
    return out


# speedup vs baseline: 1.0027x; 1.0027x over previous
import functools

import jax
import jax.numpy as jnp
from jax import lax
from jax.experimental import pallas as pl
from jax.experimental.pallas import tpu as pltpu
from jax.experimental.pallas import tpu_sc as plsc

N_NODES = 10000
H = 128
NUM_RELS = 16
E = 320000

NC = 2
NS = 16
NW = NC * NS
CHUNK = 32
K = 16
NG = 20
EPW = NG * K * CHUNK
E_PAD = NW * EPW
NPAD = 10112
RPS = NPAD // NS

NBLK = 400
GRID = N_NODES // NBLK


def _xw_body(coeff_ref, bases_ref, x_ref, xw_ref, wrel):
    @pl.when(pl.program_id(0) == 0)
    def _():
        for r in range(NUM_RELS):
            acc = coeff_ref[r, 0] * bases_ref[0]
            for b in range(1, 4):
                acc = acc + coeff_ref[r, b] * bases_ref[b]
            wrel[r] = acc

    xb = x_ref[...]
    for r in range(NUM_RELS):
        xw_ref[r] = jnp.dot(xb, wrel[r], preferred_element_type=jnp.float32)


_xw_call = pl.pallas_call(
    _xw_body,
    grid=(GRID,),
    in_specs=[
        pl.BlockSpec(memory_space=pltpu.SMEM),
        pl.BlockSpec((4, H, H), lambda i: (0, 0, 0)),
        pl.BlockSpec((NBLK, H), lambda i: (i, 0)),
    ],
    out_specs=pl.BlockSpec((NUM_RELS, NBLK, H), lambda i: (0, i, 0)),
    out_shape=jax.ShapeDtypeStruct((NUM_RELS, N_NODES, H), jnp.float32),
    scratch_shapes=[pltpu.VMEM((NUM_RELS, H, H), jnp.float32)],
)


def _agg_body(xw_hbm, gidx_hbm, didx_hbm, z2_hbm,
              agg_out,
              gidx_v, didx_v, rows0, rows1, rows2, rows3,
              agg_shared, gsem, ssem):
    c = lax.axis_index("c")
    s = lax.axis_index("s")
    w = c * NS + s
    base = s * RPS
    rows = (rows0, rows1, rows2, rows3)
    pltpu.sync_copy(z2_hbm.at[pl.ds(base, RPS)], agg_shared.at[pl.ds(base, RPS)])
    plsc.subcore_barrier()

    def group(g, carry):
        pltpu.sync_copy(gidx_hbm.at[w, g], gidx_v)
        pltpu.sync_copy(didx_hbm.at[w, g], didx_v)
        gds = {}
        for p in range(2):
            gds[p] = pltpu.async_copy(xw_hbm.at[gidx_v.at[p]], rows[p], gsem)
        sds = []
        for t in range(K):
            gds[t].wait()
            sds.append(pltpu.async_copy(
                rows[t % 4], agg_shared.at[didx_v.at[t]], ssem, add=True))
            if t >= 2:
                sds[t - 2].wait()
            if t + 2 < K:
                gds[t + 2] = pltpu.async_copy(
                    xw_hbm.at[gidx_v.at[t + 2]], rows[(t + 2) % 4], gsem)
        sds[K - 2].wait()
        sds[K - 1].wait()
        return carry

    lax.fori_loop(0, NG, group, 0)
    plsc.subcore_barrier()
    pltpu.sync_copy(agg_shared.at[pl.ds(base, RPS)],
                    agg_out.at[c, pl.ds(base, RPS)])


@functools.lru_cache(maxsize=1)
def _get_agg_call():
    return functools.partial(
        pl.kernel,
        out_type=jax.ShapeDtypeStruct((NC, NPAD, H), jnp.float32),
        mesh=plsc.VectorSubcoreMesh(core_axis_name="c", subcore_axis_name="s",
                                    num_cores=NC, num_subcores=NS),
        scratch_types=[
            pltpu.VMEM((K, CHUNK), jnp.int32),
            pltpu.VMEM((K, CHUNK), jnp.int32),
            pltpu.VMEM((CHUNK, H), jnp.float32),
            pltpu.VMEM((CHUNK, H), jnp.float32),
            pltpu.VMEM((CHUNK, H), jnp.float32),
            pltpu.VMEM((CHUNK, H), jnp.float32),
            pltpu.VMEM_SHARED((NPAD, H), jnp.float32),
            pltpu.SemaphoreType.DMA,
            pltpu.SemaphoreType.DMA,
        ],
    )(_agg_body)


def _deg_body(didx_hbm, z2_hbm, ones_hbm,
              deg_out,
              didx_v, ones_v, deg_shared, ssem):
    c = lax.axis_index("c")
    s = lax.axis_index("s")
    w = c * NS + s
    base = s * RPS
    pltpu.sync_copy(z2_hbm.at[pl.ds(base, RPS)], deg_shared.at[pl.ds(base, RPS)])
    pltpu.sync_copy(ones_hbm, ones_v)
    plsc.subcore_barrier()

    def group(g, carry):
        pltpu.sync_copy(didx_hbm.at[w, g], didx_v)
        sds = []
        for t in range(K):
            if t >= 8:
                sds[t - 8].wait()
            sds.append(pltpu.async_copy(
                ones_v, deg_shared.at[didx_v.at[t]], ssem, add=True))
        for t in range(max(0, K - 8), K):
            sds[t].wait()
        return carry

    lax.fori_loop(0, NG, group, 0)
    plsc.subcore_barrier()
    pltpu.sync_copy(deg_shared.at[pl.ds(base, RPS)],
                    deg_out.at[c, pl.ds(base, RPS)])


@functools.lru_cache(maxsize=1)
def _get_deg_call():
    return functools.partial(
        pl.kernel,
        out_type=jax.ShapeDtypeStruct((NC, NPAD, H), jnp.float32),
        mesh=plsc.VectorSubcoreMesh(core_axis_name="c", subcore_axis_name="s",
                                    num_cores=NC, num_subcores=NS),
        scratch_types=[
            pltpu.VMEM((K, CHUNK), jnp.int32),
            pltpu.VMEM((CHUNK, H), jnp.float32),
            pltpu.VMEM_SHARED((NPAD, H), jnp.float32),
            pltpu.SemaphoreType.DMA,
        ],
    )(_deg_body)


def _combine_body(agg_ref, deg_ref, x_ref, w_self_ref, out_ref):
    deg = deg_ref[0, :, 0:1] + deg_ref[1, :, 0:1]
    deg = jnp.maximum(deg, 1.0)
    agg = agg_ref[0] + agg_ref[1]
    out_ref[...] = agg / deg + jnp.dot(
        x_ref[...], w_self_ref[...], preferred_element_type=jnp.float32)


_combine_call = pl.pallas_call(
    _combine_body,
    grid=(GRID,),
    in_specs=[
        pl.BlockSpec((NC, NBLK, H), lambda i: (0, i, 0)),
        pl.BlockSpec((NC, NBLK, H), lambda i: (0, i, 0)),
        pl.BlockSpec((NBLK, H), lambda i: (i, 0)),
        pl.BlockSpec((H, H), lambda i: (0, 0)),
    ],
    out_specs=pl.BlockSpec((NBLK, H), lambda i: (i, 0)),
    out_shape=jax.ShapeDtypeStruct((N_NODES, H), jnp.float32),
)


def kernel(h, edge_index, edge_type, embed_table, bases, coeff, w_self):
    x = jnp.take(embed_table, h, axis=0)
    xw = _xw_call(coeff, bases, x)
    xw_flat = xw.reshape(NUM_RELS * N_NODES, H)

    src = edge_index[0]
    dst = edge_index[1]
    flat = edge_type * N_NODES + src
    pad = E_PAD - E
    gidx = jnp.concatenate(
        [flat, jnp.zeros((pad,), jnp.int32)]).reshape(NW, NG, K, CHUNK)
    didx_flat = jnp.concatenate([dst, jnp.full((pad,), N_NODES, jnp.int32)])
    didx = didx_flat.reshape(NW, NG, K, CHUNK)
    z2 = jnp.zeros((NPAD, H), jnp.float32)
    ones_in = jnp.ones((CHUNK, H), jnp.float32)

    agg_part = _get_agg_call()(xw_flat, gidx, didx, z2)
    deg_part = _get_deg_call()(didx, z2, ones_in)
    out = _combine_call(agg_part, deg_part, x, w_self)
    return out
